# baseline (speedup 1.0000x reference)
import jax
import jax.numpy as jnp
from jax.experimental import pallas as pl
from jax.experimental.pallas import tpu as pltpu

_INPUT_SIZE = 128
_SEQ_LENGTH = 16
_VOCAB = 64
_TN = 512


def _fused_body(x_ref, w_ref, seq_ref, score_ref):
    logits = jnp.dot(x_ref[:], w_ref[:], preferred_element_type=jnp.float32)
    lane = jax.lax.broadcasted_iota(jnp.int32, (_TN, _VOCAB), 1)
    idx_cols = []
    max_cols = []
    for t in range(_SEQ_LENGTH):
        lt = logits[:, t * _VOCAB:(t + 1) * _VOCAB]
        m = jnp.max(lt, axis=1, keepdims=True)
        idx = jnp.min(jnp.where(lt == m, lane, _VOCAB), axis=1)
        idx_cols.append(idx[None, :])
        max_cols.append(m)
    seq_ref[:, :] = jnp.concatenate(idx_cols, axis=0)
    score_ref[:] = jnp.sum(jnp.concatenate(max_cols, axis=1), axis=1)


def kernel(observation, W):
    batch = observation.shape[0]
    ipo = observation.shape[1] // _INPUT_SIZE
    n = batch * ipo
    x = observation.reshape(n, _INPUT_SIZE)
    w2 = W.reshape(_INPUT_SIZE, _SEQ_LENGTH * _VOCAB)

    grid = (n // _TN,)
    seq_t, scores = pl.pallas_call(
        _fused_body,
        grid=grid,
        in_specs=[
            pl.BlockSpec((_TN, _INPUT_SIZE), lambda i: (i, 0)),
            pl.BlockSpec((_INPUT_SIZE, _SEQ_LENGTH * _VOCAB), lambda i: (0, 0)),
        ],
        out_specs=[
            pl.BlockSpec((_SEQ_LENGTH, _TN), lambda i: (0, i)),
            pl.BlockSpec((_TN,), lambda i: (i,)),
        ],
        out_shape=[
            jax.ShapeDtypeStruct((_SEQ_LENGTH, n), jnp.int32),
            jax.ShapeDtypeStruct((n,), jnp.float32),
        ],
        compiler_params=pltpu.CompilerParams(
            dimension_semantics=("arbitrary",),
        ),
    )(x, w2)

    seqs = seq_t.T
    seq_supp_batch = seqs.reshape(batch, ipo, _SEQ_LENGTH)
    length_supp_batch = jnp.full((batch, ipo), _SEQ_LENGTH, dtype=jnp.int32)
    score_batch = scores.reshape(batch, ipo)
    return seq_supp_batch, length_supp_batch, score_batch


# speedup vs baseline: 3.1152x; 3.1152x over previous
import jax
import jax.numpy as jnp
from jax.experimental import pallas as pl
from jax.experimental.pallas import tpu as pltpu

_INPUT_SIZE = 128
_SEQ_LENGTH = 16
_VOCAB = 64
_TN = 512


def _fused_body(w2t_ref, xt_ref, seq_ref, score_ref):
    logits_t = jnp.dot(w2t_ref[:], xt_ref[:],
                       preferred_element_type=jnp.float32)
    r = logits_t.reshape(_SEQ_LENGTH, _VOCAB, _TN)
    m = jnp.max(r, axis=1)
    sub_f = jax.lax.broadcasted_iota(
        jnp.int32, (_SEQ_LENGTH, _VOCAB, _TN), 1).astype(jnp.float32)
    idxf = jnp.min(jnp.where(r == m[:, None, :], sub_f, float(_VOCAB)),
                   axis=1)
    seq_ref[:, :] = idxf.astype(jnp.int32)
    score_ref[:, :] = jnp.sum(m, axis=0, keepdims=True)


def kernel(observation, W):
    batch = observation.shape[0]
    ipo = observation.shape[1] // _INPUT_SIZE
    n = batch * ipo
    xt = observation.reshape(batch * ipo, _INPUT_SIZE).T
    w2t = W.reshape(_INPUT_SIZE, _SEQ_LENGTH * _VOCAB).T

    grid = (n // _TN,)
    seq_t, score_t = pl.pallas_call(
        _fused_body,
        grid=grid,
        in_specs=[
            pl.BlockSpec((_SEQ_LENGTH * _VOCAB, _INPUT_SIZE), lambda i: (0, 0)),
            pl.BlockSpec((_INPUT_SIZE, _TN), lambda i: (0, i)),
        ],
        out_specs=[
            pl.BlockSpec((_SEQ_LENGTH, _TN), lambda i: (0, i)),
            pl.BlockSpec((1, _TN), lambda i: (0, i)),
        ],
        out_shape=[
            jax.ShapeDtypeStruct((_SEQ_LENGTH, n), jnp.int32),
            jax.ShapeDtypeStruct((1, n), jnp.float32),
        ],
        compiler_params=pltpu.CompilerParams(
            dimension_semantics=("arbitrary",),
        ),
    )(w2t, xt)

    seq_supp_batch = seq_t.T.reshape(batch, ipo, _SEQ_LENGTH)
    length_supp_batch = jnp.full((batch, ipo), _SEQ_LENGTH, dtype=jnp.int32)
    score_batch = score_t.reshape(batch, ipo)
    return seq_supp_batch, length_supp_batch, score_batch


# speedup vs baseline: 4.2173x; 1.3538x over previous
import jax
import jax.numpy as jnp
from jax.experimental import pallas as pl
from jax.experimental.pallas import tpu as pltpu

_INPUT_SIZE = 128
_SEQ_LENGTH = 16
_VOCAB = 64
_TN = 512


def _fused_body(w2t_ref, x_ref, seq_ref, score_ref):
    logits_t = jax.lax.dot_general(
        w2t_ref[:], x_ref[:], (((1,), (1,)), ((), ())),
        preferred_element_type=jnp.float32)
    r = logits_t.reshape(_SEQ_LENGTH, _VOCAB, _TN)
    m = jnp.max(r, axis=1)
    sub_f = jax.lax.broadcasted_iota(
        jnp.int32, (_SEQ_LENGTH, _VOCAB, _TN), 1).astype(jnp.float32)
    idxf = jnp.min(jnp.where(r == m[:, None, :], sub_f, float(_VOCAB)),
                   axis=1)
    seq_ref[:, :] = idxf.astype(jnp.int32)
    score_ref[:, :] = jnp.sum(m, axis=0, keepdims=True)


def kernel(observation, W):
    batch = observation.shape[0]
    ipo = observation.shape[1] // _INPUT_SIZE
    n = batch * ipo
    x = observation.reshape(batch * ipo, _INPUT_SIZE)
    w2t = W.reshape(_INPUT_SIZE, _SEQ_LENGTH * _VOCAB).T

    grid = (n // _TN,)
    seq_t, score_t = pl.pallas_call(
        _fused_body,
        grid=grid,
        in_specs=[
            pl.BlockSpec((_SEQ_LENGTH * _VOCAB, _INPUT_SIZE), lambda i: (0, 0)),
            pl.BlockSpec((_TN, _INPUT_SIZE), lambda i: (i, 0)),
        ],
        out_specs=[
            pl.BlockSpec((_SEQ_LENGTH, _TN), lambda i: (0, i)),
            pl.BlockSpec((1, _TN), lambda i: (0, i)),
        ],
        out_shape=[
            jax.ShapeDtypeStruct((_SEQ_LENGTH, n), jnp.int32),
            jax.ShapeDtypeStruct((1, n), jnp.float32),
        ],
        compiler_params=pltpu.CompilerParams(
            dimension_semantics=("arbitrary",),
        ),
    )(w2t, x)

    seq_supp_batch = seq_t.T.reshape(batch, ipo, _SEQ_LENGTH)
    length_supp_batch = jnp.full((batch, ipo), _SEQ_LENGTH, dtype=jnp.int32)
    score_batch = score_t.reshape(batch, ipo)
    return seq_supp_batch, length_supp_batch, score_batch


# speedup vs baseline: 5.2426x; 1.2431x over previous
import jax
import jax.numpy as jnp
from jax.experimental import pallas as pl
from jax.experimental.pallas import tpu as pltpu

_INPUT_SIZE = 128
_SEQ_LENGTH = 16
_VOCAB = 64
_TN = 1024


def _fused_body(w2t_ref, x_ref, seq_ref, score_ref):
    logits_t = jax.lax.dot_general(
        w2t_ref[:], x_ref[:], (((1,), (1,)), ((), ())),
        preferred_element_type=jnp.float32)
    r = logits_t.reshape(_SEQ_LENGTH, _VOCAB, _TN)
    m = jnp.max(r, axis=1)
    sub_f = jax.lax.broadcasted_iota(
        jnp.int32, (_SEQ_LENGTH, _VOCAB, _TN), 1).astype(jnp.float32)
    idxf = jnp.min(jnp.where(r == m[:, None, :], sub_f, float(_VOCAB)),
                   axis=1)
    seq_ref[:, :] = idxf.astype(jnp.int32)
    score_ref[:, :] = jnp.sum(m, axis=0, keepdims=True)


def kernel(observation, W):
    batch = observation.shape[0]
    ipo = observation.shape[1] // _INPUT_SIZE
    n = batch * ipo
    x = observation.reshape(batch * ipo, _INPUT_SIZE)
    w2t = W.reshape(_INPUT_SIZE, _SEQ_LENGTH * _VOCAB).T

    grid = (n // _TN,)
    seq_t, score_t = pl.pallas_call(
        _fused_body,
        grid=grid,
        in_specs=[
            pl.BlockSpec((_SEQ_LENGTH * _VOCAB, _INPUT_SIZE), lambda i: (0, 0)),
            pl.BlockSpec((_TN, _INPUT_SIZE), lambda i: (i, 0)),
        ],
        out_specs=[
            pl.BlockSpec((_SEQ_LENGTH, _TN), lambda i: (0, i)),
            pl.BlockSpec((1, _TN), lambda i: (0, i)),
        ],
        out_shape=[
            jax.ShapeDtypeStruct((_SEQ_LENGTH, n), jnp.int32),
            jax.ShapeDtypeStruct((1, n), jnp.float32),
        ],
        compiler_params=pltpu.CompilerParams(
            dimension_semantics=("arbitrary",),
        ),
    )(w2t, x)

    seq_supp_batch = seq_t.T.reshape(batch, ipo, _SEQ_LENGTH)
    length_supp_batch = jnp.full((batch, ipo), _SEQ_LENGTH, dtype=jnp.int32)
    score_batch = score_t.reshape(batch, ipo)
    return seq_supp_batch, length_supp_batch, score_batch
